# baseline (speedup 1.0000x reference)
import jax
import jax.numpy as jnp
from jax.experimental import pallas as pl
from jax.experimental.pallas import tpu as pltpu


def _gcn_body(feat_ref, adj_top_ref, adj_bot_ref, w_ref, b_ref, out_ref, support_ref):
    i = pl.program_id(0)
    h = adj_top_ref.shape[0]

    @pl.when(i == 0)
    def _():
        support_ref[...] = jnp.dot(
            feat_ref[...], w_ref[...], preferred_element_type=jnp.float32
        )

    sup = support_ref[...]
    acc_t = jnp.dot(adj_top_ref[...], sup, preferred_element_type=jnp.float32)
    acc_b = jnp.dot(adj_bot_ref[...], sup, preferred_element_type=jnp.float32)
    base = i * 2 * h
    out_ref[pl.ds(0, h), :] = (
        jnp.maximum(acc_t + b_ref[...], 0.0) + feat_ref[pl.ds(base, h), :]
    )
    out_ref[pl.ds(h, h), :] = (
        jnp.maximum(acc_b + b_ref[...], 0.0) + feat_ref[pl.ds(base + h, h), :]
    )


def kernel(features, adj_norm, weight, bias):
    n, d = features.shape
    bm = 400
    h = bm // 2
    assert n % bm == 0
    bias2 = bias.reshape(1, d)

    return pl.pallas_call(
        _gcn_body,
        grid=(n // bm,),
        in_specs=[
            pl.BlockSpec((n, d), lambda i: (0, 0)),
            pl.BlockSpec((h, n), lambda i: (2 * i, 0)),
            pl.BlockSpec((h, n), lambda i: (2 * i + 1, 0)),
            pl.BlockSpec((d, d), lambda i: (0, 0)),
            pl.BlockSpec((1, d), lambda i: (0, 0)),
        ],
        out_specs=pl.BlockSpec((bm, d), lambda i: (i, 0)),
        out_shape=jax.ShapeDtypeStruct((n, d), jnp.float32),
        scratch_shapes=[pltpu.VMEM((n, d), jnp.float32)],
    )(features, adj_norm, adj_norm, weight, bias2)


# speedup vs baseline: 1.0175x; 1.0175x over previous
import jax
import jax.numpy as jnp
from jax.experimental import pallas as pl
from jax.experimental.pallas import tpu as pltpu


def _gcn_body(feat_ref, adj_ref, w_ref, b_ref, out_ref, support_ref):
    i = pl.program_id(0)
    bm = out_ref.shape[0]

    @pl.when(i == 0)
    def _():
        support_ref[...] = jnp.dot(
            feat_ref[...], w_ref[...], preferred_element_type=jnp.float32
        )

    acc = jnp.dot(adj_ref[...], support_ref[...], preferred_element_type=jnp.float32)
    feat_blk = feat_ref[pl.ds(i * bm, bm), :]
    out_ref[...] = jnp.maximum(acc + b_ref[...], 0.0) + feat_blk


def kernel(features, adj_norm, weight, bias):
    n, d = features.shape
    bm = 400
    assert n % bm == 0
    bias2 = bias.reshape(1, d)

    return pl.pallas_call(
        _gcn_body,
        grid=(n // bm,),
        in_specs=[
            pl.BlockSpec((n, d), lambda i: (0, 0)),
            pl.BlockSpec((bm, n), lambda i: (i, 0)),
            pl.BlockSpec((d, d), lambda i: (0, 0)),
            pl.BlockSpec((1, d), lambda i: (0, 0)),
        ],
        out_specs=pl.BlockSpec((bm, d), lambda i: (i, 0)),
        out_shape=jax.ShapeDtypeStruct((n, d), jnp.float32),
        scratch_shapes=[pltpu.VMEM((n, d), jnp.float32)],
    )(features, adj_norm, weight, bias2)
